# Initial kernel scaffold; baseline (speedup 1.0000x reference)
import functools

import jax
import jax.numpy as jnp
from jax import lax
from jax.experimental import pallas as pl
from jax.experimental.pallas import tpu as pltpu
from jax.experimental.pallas import tpu_sc as plsc

NC = 2
NS = 16
L = 16


@functools.partial(jax.jit, static_argnames=("n_pad", "e_per_tile", "k"))
def _sc_degree(dst, zeros_n, ones_k, *, n_pad, e_per_tile, k):
  rows_per_tile = n_pad // NS
  mesh = plsc.VectorSubcoreMesh(core_axis_name="c", subcore_axis_name="s")

  @functools.partial(
      pl.kernel,
      mesh=mesh,
      out_type=jax.ShapeDtypeStruct((NC, n_pad), jnp.float32),
      scratch_types=[
          pltpu.VMEM((k,), jnp.int32),
          pltpu.VMEM((k,), jnp.float32),
          pltpu.VMEM_SHARED((n_pad,), jnp.float32),
      ],
  )
  def body(dst_hbm, zeros_hbm, ones_hbm, degp_hbm, idx_v, ones_v, acc_sh):
    c = lax.axis_index("c")
    s = lax.axis_index("s")
    sl = pl.ds(s * rows_per_tile, rows_per_tile)
    pltpu.sync_copy(zeros_hbm.at[sl], acc_sh.at[sl])
    pltpu.sync_copy(ones_hbm, ones_v)
    plsc.subcore_barrier()
    base = (c * NS + s) * e_per_tile

    def chunk(i, carry):
      pltpu.sync_copy(dst_hbm.at[pl.ds(base + i * k, k)], idx_v)
      pltpu.sync_copy(ones_v, acc_sh.at[idx_v], add=True)
      return carry

    lax.fori_loop(0, e_per_tile // k, chunk, 0)
    plsc.subcore_barrier()
    pltpu.sync_copy(acc_sh.at[sl], degp_hbm.at[c, sl])

  return body(dst, zeros_n, ones_k)


@functools.partial(jax.jit, static_argnames=("n_pad", "e_per_tile", "k"))
def _sc_aggregate(src, dst, hs, zeros_nh, *, n_pad, e_per_tile, k):
  h_dim = hs.shape[1]
  rows_per_tile = n_pad // NS
  mesh = plsc.VectorSubcoreMesh(core_axis_name="c", subcore_axis_name="s")

  @functools.partial(
      pl.kernel,
      mesh=mesh,
      out_type=jax.ShapeDtypeStruct((NC, n_pad, h_dim), jnp.float32),
      scratch_types=[
          pltpu.VMEM((k,), jnp.int32),
          pltpu.VMEM((k,), jnp.int32),
          pltpu.VMEM((k, h_dim), jnp.float32),
          pltpu.VMEM_SHARED((n_pad, h_dim), jnp.float32),
          pltpu.SemaphoreType.DMA,
      ],
  )
  def body(src_hbm, dst_hbm, hs_hbm, zeros_hbm, aggp_hbm,
           sidx_v, didx_v, rows_v, acc_sh, sem):
    c = lax.axis_index("c")
    s = lax.axis_index("s")
    sl = pl.ds(s * rows_per_tile, rows_per_tile)
    pltpu.sync_copy(zeros_hbm.at[sl], acc_sh.at[sl])
    plsc.subcore_barrier()
    base = (c * NS + s) * e_per_tile

    def chunk(i, carry):
      off = base + i * k
      pltpu.sync_copy(src_hbm.at[pl.ds(off, k)], sidx_v)
      pltpu.sync_copy(dst_hbm.at[pl.ds(off, k)], didx_v)
      pltpu.async_copy(hs_hbm.at[sidx_v], rows_v, sem).wait()
      pltpu.sync_copy(rows_v, acc_sh.at[didx_v], add=True)
      return carry

    lax.fori_loop(0, e_per_tile // k, chunk, 0)
    plsc.subcore_barrier()
    pltpu.sync_copy(acc_sh.at[sl], aggp_hbm.at[c, sl])

  return body(src, dst, hs, zeros_nh)


def _tc_scaled_linear(x, w1, deg0, deg1):
  n, d = x.shape
  h_dim = w1.shape[1]
  bn = 1000

  def body(x_ref, w_ref, d0_ref, d1_ref, hs_ref):
    deg = d0_ref[...] + d1_ref[...] + 1.0
    dinv = lax.rsqrt(deg)
    h = jnp.dot(x_ref[...], w_ref[...], preferred_element_type=jnp.float32)
    hs_ref[...] = h * dinv[:, None]

  return pl.pallas_call(
      body,
      grid=(n // bn,),
      in_specs=[
          pl.BlockSpec((bn, d), lambda i: (i, 0)),
          pl.BlockSpec((d, h_dim), lambda i: (0, 0)),
          pl.BlockSpec((bn,), lambda i: (i,)),
          pl.BlockSpec((bn,), lambda i: (i,)),
      ],
      out_specs=pl.BlockSpec((bn, h_dim), lambda i: (i, 0)),
      out_shape=jax.ShapeDtypeStruct((n, h_dim), jnp.float32),
  )(x, w1, deg0, deg1)


def _tc_final(agg0, agg1, hs, deg0, deg1, b1, wl, bl):
  n, h_dim = hs.shape
  c_dim = wl.shape[1]
  bn = 1000

  def body(a0_ref, a1_ref, hs_ref, d0_ref, d1_ref, b1_ref, wl_ref, bl_ref,
           out_ref):
    deg = d0_ref[...] + d1_ref[...] + 1.0
    dinv = lax.rsqrt(deg)
    pre = (a0_ref[...] + a1_ref[...] + hs_ref[...]) * dinv[:, None] + b1_ref[...]
    act = jnp.maximum(pre, 0.0)
    logits = jnp.dot(act, wl_ref[...], preferred_element_type=jnp.float32)
    logits = logits + bl_ref[...]
    out_ref[...] = 1.0 / (1.0 + jnp.exp(-logits))

  return pl.pallas_call(
      body,
      grid=(n // bn,),
      in_specs=[
          pl.BlockSpec((bn, h_dim), lambda i: (i, 0)),
          pl.BlockSpec((bn, h_dim), lambda i: (i, 0)),
          pl.BlockSpec((bn, h_dim), lambda i: (i, 0)),
          pl.BlockSpec((bn,), lambda i: (i,)),
          pl.BlockSpec((bn,), lambda i: (i,)),
          pl.BlockSpec((1, h_dim), lambda i: (0, 0)),
          pl.BlockSpec((h_dim, c_dim), lambda i: (0, 0)),
          pl.BlockSpec((1, c_dim), lambda i: (0, 0)),
      ],
      out_specs=pl.BlockSpec((bn, c_dim), lambda i: (i, 0)),
      out_shape=jax.ShapeDtypeStruct((n, c_dim), jnp.float32),
  )(agg0, agg1, hs, deg0, deg1, b1, wl, bl)


def kernel(x_muons, edge_index_muons, W1, b1, Wl, bl, generate_jets=0):
  n, _ = x_muons.shape
  h_dim = W1.shape[1]
  e = edge_index_muons.shape[1]

  n_pad = ((n + (NS * L) - 1) // (NS * L)) * (NS * L)
  e_per_tile = e // (NC * NS)
  k = 2000

  src = edge_index_muons[0]
  dst = edge_index_muons[1]

  zeros_n = jnp.zeros((n_pad,), jnp.float32)
  ones_k = jnp.ones((k,), jnp.float32)
  zeros_nh = jnp.zeros((n_pad, h_dim), jnp.float32)

  degp = _sc_degree(dst, zeros_n, ones_k,
                    n_pad=n_pad, e_per_tile=e_per_tile, k=k)
  deg0 = degp[0, :n]
  deg1 = degp[1, :n]

  hs = _tc_scaled_linear(x_muons, W1, deg0, deg1)

  aggp = _sc_aggregate(src, dst, hs, zeros_nh,
                       n_pad=n_pad, e_per_tile=e_per_tile, k=k)

  out = _tc_final(aggp[0, :n], aggp[1, :n], hs, deg0, deg1,
                  b1.reshape(1, h_dim), Wl, bl.reshape(1, -1))
  return out


# speedup vs baseline: 57.3359x; 57.3359x over previous
import functools

import jax
import jax.numpy as jnp
from jax import lax
from jax.experimental import pallas as pl
from jax.experimental.pallas import tpu as pltpu
from jax.experimental.pallas import tpu_sc as plsc

NC = 2
NS = 16
L = 16


@functools.partial(jax.jit, static_argnames=("n_pad", "e_per_tile", "k"))
def _sc_degree(dst, zeros_n, ones_k, *, n_pad, e_per_tile, k):
  rows_per_tile = n_pad // NS
  mesh = plsc.VectorSubcoreMesh(core_axis_name="c", subcore_axis_name="s")

  @functools.partial(
      pl.kernel,
      mesh=mesh,
      out_type=jax.ShapeDtypeStruct((NC, n_pad), jnp.float32),
      scratch_types=[
          pltpu.VMEM((k,), jnp.int32),
          pltpu.VMEM((k,), jnp.float32),
          pltpu.VMEM_SHARED((n_pad,), jnp.float32),
      ],
  )
  def body(dst_hbm, zeros_hbm, ones_hbm, degp_hbm, idx_v, ones_v, acc_sh):
    c = lax.axis_index("c")
    s = lax.axis_index("s")
    sl = pl.ds(s * rows_per_tile, rows_per_tile)
    pltpu.sync_copy(zeros_hbm.at[sl], acc_sh.at[sl])
    pltpu.sync_copy(ones_hbm, ones_v)
    plsc.subcore_barrier()
    base = (c * NS + s) * e_per_tile

    def chunk(i, carry):
      pltpu.sync_copy(dst_hbm.at[pl.ds(base + i * k, k)], idx_v)
      pltpu.sync_copy(ones_v, acc_sh.at[idx_v], add=True)
      return carry

    lax.fori_loop(0, e_per_tile // k, chunk, 0)
    plsc.subcore_barrier()
    pltpu.sync_copy(acc_sh.at[sl], degp_hbm.at[c, sl])

  return body(dst, zeros_n, ones_k)


@functools.partial(jax.jit, static_argnames=("n_pad", "e_per_tile", "k"))
def _sc_aggregate(src, dst, hs, zeros_nh, *, n_pad, e_per_tile, k):
  h_dim = hs.shape[1]
  rows_per_tile = n_pad // NS
  mesh = plsc.VectorSubcoreMesh(core_axis_name="c", subcore_axis_name="s")

  @functools.partial(
      pl.kernel,
      mesh=mesh,
      out_type=jax.ShapeDtypeStruct((NC, n_pad, h_dim), jnp.float32),
      compiler_params=pltpu.CompilerParams(use_tc_tiling_on_sc=False),
      scratch_types=[
          pltpu.VMEM((k,), jnp.int32),
          pltpu.VMEM((k,), jnp.int32),
          pltpu.VMEM((k, h_dim), jnp.float32),
          pltpu.VMEM_SHARED((n_pad, h_dim), jnp.float32),
          pltpu.SemaphoreType.DMA,
      ],
  )
  def body(src_hbm, dst_hbm, hs_hbm, zeros_hbm, aggp_hbm,
           sidx_v, didx_v, rows_v, acc_sh, sem):
    c = lax.axis_index("c")
    s = lax.axis_index("s")
    sl = pl.ds(s * rows_per_tile, rows_per_tile)
    pltpu.sync_copy(zeros_hbm.at[sl], acc_sh.at[sl])
    plsc.subcore_barrier()
    base = (c * NS + s) * e_per_tile

    def chunk(i, carry):
      off = base + i * k
      pltpu.sync_copy(src_hbm.at[pl.ds(off, k)], sidx_v)
      pltpu.sync_copy(dst_hbm.at[pl.ds(off, k)], didx_v)
      pltpu.async_copy(hs_hbm.at[sidx_v], rows_v, sem).wait()
      pltpu.sync_copy(rows_v, acc_sh.at[didx_v], add=True)
      return carry

    lax.fori_loop(0, e_per_tile // k, chunk, 0)
    plsc.subcore_barrier()
    pltpu.sync_copy(acc_sh.at[sl], aggp_hbm.at[c, sl])

  return body(src, dst, hs, zeros_nh)


def _tc_scaled_linear(x, w1, deg0, deg1):
  n, d = x.shape
  h_dim = w1.shape[1]
  bn = 1000

  def body(x_ref, w_ref, d0_ref, d1_ref, hs_ref):
    deg = d0_ref[...] + d1_ref[...] + 1.0
    dinv = lax.rsqrt(deg)
    h = jnp.dot(x_ref[...], w_ref[...], preferred_element_type=jnp.float32)
    hs_ref[...] = h * dinv

  return pl.pallas_call(
      body,
      grid=(n // bn,),
      in_specs=[
          pl.BlockSpec((bn, d), lambda i: (i, 0)),
          pl.BlockSpec((d, h_dim), lambda i: (0, 0)),
          pl.BlockSpec((bn, 1), lambda i: (i, 0)),
          pl.BlockSpec((bn, 1), lambda i: (i, 0)),
      ],
      out_specs=pl.BlockSpec((bn, h_dim), lambda i: (i, 0)),
      out_shape=jax.ShapeDtypeStruct((n, h_dim), jnp.float32),
  )(x, w1, deg0, deg1)


def _tc_final(agg0, agg1, hs, deg0, deg1, b1, wl, bl):
  n, h_dim = hs.shape
  c_dim = wl.shape[1]
  bn = 1000

  def body(a0_ref, a1_ref, hs_ref, d0_ref, d1_ref, b1_ref, wl_ref, bl_ref,
           out_ref):
    deg = d0_ref[...] + d1_ref[...] + 1.0
    dinv = lax.rsqrt(deg)
    pre = (a0_ref[...] + a1_ref[...] + hs_ref[...]) * dinv + b1_ref[...]
    act = jnp.maximum(pre, 0.0)
    logits = jnp.dot(act, wl_ref[...], preferred_element_type=jnp.float32)
    logits = logits + bl_ref[...]
    out_ref[...] = 1.0 / (1.0 + jnp.exp(-logits))

  return pl.pallas_call(
      body,
      grid=(n // bn,),
      in_specs=[
          pl.BlockSpec((bn, h_dim), lambda i: (i, 0)),
          pl.BlockSpec((bn, h_dim), lambda i: (i, 0)),
          pl.BlockSpec((bn, h_dim), lambda i: (i, 0)),
          pl.BlockSpec((bn, 1), lambda i: (i, 0)),
          pl.BlockSpec((bn, 1), lambda i: (i, 0)),
          pl.BlockSpec((1, h_dim), lambda i: (0, 0)),
          pl.BlockSpec((h_dim, c_dim), lambda i: (0, 0)),
          pl.BlockSpec((1, c_dim), lambda i: (0, 0)),
      ],
      out_specs=pl.BlockSpec((bn, c_dim), lambda i: (i, 0)),
      out_shape=jax.ShapeDtypeStruct((n, c_dim), jnp.float32),
  )(agg0, agg1, hs, deg0, deg1, b1, wl, bl)


def kernel(x_muons, edge_index_muons, W1, b1, Wl, bl, generate_jets=0):
  n, _ = x_muons.shape
  h_dim = W1.shape[1]
  e = edge_index_muons.shape[1]

  n_pad = ((n + (NS * L) - 1) // (NS * L)) * (NS * L)
  e_per_tile = e // (NC * NS)
  k = 2000

  src = edge_index_muons[0]
  dst = edge_index_muons[1]

  zeros_n = jnp.zeros((n_pad,), jnp.float32)
  ones_k = jnp.ones((k,), jnp.float32)
  zeros_nh = jnp.zeros((n_pad, h_dim), jnp.float32)

  degp = _sc_degree(dst, zeros_n, ones_k,
                    n_pad=n_pad, e_per_tile=e_per_tile, k=k)
  deg0 = degp[0, :n].reshape(n, 1)
  deg1 = degp[1, :n].reshape(n, 1)

  hs = _tc_scaled_linear(x_muons, W1, deg0, deg1)

  aggp = _sc_aggregate(src, dst, hs, zeros_nh,
                       n_pad=n_pad, e_per_tile=e_per_tile, k=k)

  out = _tc_final(aggp[0, :n], aggp[1, :n], hs, deg0, deg1,
                  b1.reshape(1, h_dim), Wl, bl.reshape(1, -1))
  return out


# speedup vs baseline: 90.9740x; 1.5867x over previous
import functools

import jax
import jax.numpy as jnp
from jax import lax
from jax.experimental import pallas as pl
from jax.experimental.pallas import tpu as pltpu
from jax.experimental.pallas import tpu_sc as plsc

NC = 2
NS = 16
L = 16
NPAD = 10240
RPT = NPAD // NS
KCH = 2000


def _newton_rsqrt(d):
  i = lax.bitcast_convert_type(d, jnp.int32)
  i = 0x5F3759DF - lax.shift_right_arithmetic(i, 1)
  y = lax.bitcast_convert_type(i, jnp.float32)
  for _ in range(4):
    y = y * (1.5 - 0.5 * d * y * y)
  return y


_SPLAT_DNUMS = lax.GatherDimensionNumbers(
    offset_dims=(), collapsed_slice_dims=(0,), start_index_map=(0,))


def _splat(vec, jj):
  idx = jnp.full((L, 1), jj, jnp.int32)
  return lax.gather(vec, idx, _SPLAT_DNUMS, (1,),
                    mode=lax.GatherScatterMode.PROMISE_IN_BOUNDS)


@functools.partial(jax.jit, static_argnames=("e_per_tile",))
def _sc_degree(edge_index, zeros_n, ones_k, *, e_per_tile):
  mesh = plsc.VectorSubcoreMesh(core_axis_name="c", subcore_axis_name="s")

  @functools.partial(
      pl.kernel,
      mesh=mesh,
      out_type=jax.ShapeDtypeStruct((NC, NPAD), jnp.float32),
      compiler_params=pltpu.CompilerParams(use_tc_tiling_on_sc=False),
      scratch_types=[
          pltpu.VMEM((KCH,), jnp.int32),
          pltpu.VMEM((KCH,), jnp.float32),
          pltpu.VMEM_SHARED((NPAD,), jnp.float32),
      ],
  )
  def body(ei_hbm, zeros_hbm, ones_hbm, degp_hbm, idx_v, ones_v, acc_sh):
    c = lax.axis_index("c")
    s = lax.axis_index("s")
    sl = pl.ds(s * RPT, RPT)
    pltpu.sync_copy(zeros_hbm.at[sl], acc_sh.at[sl])
    pltpu.sync_copy(ones_hbm, ones_v)
    plsc.subcore_barrier()
    base = (c * NS + s) * e_per_tile

    def chunk(i, carry):
      pltpu.sync_copy(ei_hbm.at[1, pl.ds(base + i * KCH, KCH)], idx_v)
      pltpu.sync_copy(ones_v, acc_sh.at[idx_v], add=True)
      return carry

    lax.fori_loop(0, e_per_tile // KCH, chunk, 0)
    plsc.subcore_barrier()
    pltpu.sync_copy(acc_sh.at[sl], degp_hbm.at[c, sl])

  return body(edge_index, zeros_n, ones_k)


@functools.partial(jax.jit, static_argnames=("e_per_tile",))
def _sc_aggregate(edge_index, h, degp, zeros_nh, *, e_per_tile):
  mesh = plsc.VectorSubcoreMesh(core_axis_name="c", subcore_axis_name="s")

  @functools.partial(
      pl.kernel,
      mesh=mesh,
      out_type=(jax.ShapeDtypeStruct((NC, NPAD, L), jnp.float32),
                jax.ShapeDtypeStruct((NPAD, L), jnp.float32)),
      compiler_params=pltpu.CompilerParams(use_tc_tiling_on_sc=False),
      scratch_types=[
          pltpu.VMEM((RPT,), jnp.float32),
          pltpu.VMEM((RPT,), jnp.float32),
          pltpu.VMEM((RPT, L), jnp.float32),
          pltpu.VMEM((RPT, L), jnp.float32),
          pltpu.VMEM((KCH,), jnp.int32),
          pltpu.VMEM((KCH,), jnp.int32),
          pltpu.VMEM((KCH, L), jnp.float32),
          pltpu.VMEM_SHARED((NPAD, L), jnp.float32),
          pltpu.SemaphoreType.DMA,
      ],
  )
  def body(ei_hbm, h_hbm, degp_hbm, zeros_hbm, aggp_hbm, hs_hbm,
           d1_v, dinv_v, h_v, hs_v, sidx_v, didx_v, rows_v, acc_sh, sem):
    c = lax.axis_index("c")
    s = lax.axis_index("s")
    sl = pl.ds(s * RPT, RPT)

    pltpu.sync_copy(degp_hbm.at[0, sl], dinv_v)
    pltpu.sync_copy(degp_hbm.at[1, sl], d1_v)
    pltpu.sync_copy(h_hbm.at[sl], h_v)

    def newton(g, carry):
      gsl = pl.ds(g * L, L)
      deg = dinv_v[gsl] + d1_v[gsl] + 1.0
      dinv_v[gsl] = _newton_rsqrt(deg)
      return carry

    lax.fori_loop(0, RPT // L, newton, 0)

    def scale_hs(g, carry):
      dchunk = dinv_v[pl.ds(g * L, L)]
      for jj in range(L):
        j = g * L + jj
        hs_v[j, :] = h_v[j, :] * _splat(dchunk, jj)
      return carry

    lax.fori_loop(0, RPT // L, scale_hs, 0)
    pltpu.sync_copy(hs_v, hs_hbm.at[sl])

    @pl.when(c == 0)
    def _():
      pltpu.sync_copy(hs_v, acc_sh.at[sl])

    @pl.when(c != 0)
    def _():
      pltpu.sync_copy(zeros_hbm.at[sl], acc_sh.at[sl])

    plsc.subcore_barrier()

    base = (c * NS + s) * e_per_tile

    def chunk(i, carry):
      off = base + i * KCH
      pltpu.sync_copy(ei_hbm.at[0, pl.ds(off, KCH)], sidx_v)
      pltpu.sync_copy(ei_hbm.at[1, pl.ds(off, KCH)], didx_v)
      pltpu.async_copy(hs_hbm.at[sidx_v], rows_v, sem).wait()
      pltpu.sync_copy(rows_v, acc_sh.at[didx_v], add=True)
      return carry

    lax.fori_loop(0, e_per_tile // KCH, chunk, 0)
    plsc.subcore_barrier()

    pltpu.sync_copy(acc_sh.at[sl], h_v)

    def scale_out(g, carry):
      dchunk = dinv_v[pl.ds(g * L, L)]
      for jj in range(L):
        j = g * L + jj
        h_v[j, :] = h_v[j, :] * _splat(dchunk, jj)
      return carry

    lax.fori_loop(0, RPT // L, scale_out, 0)
    pltpu.sync_copy(h_v, aggp_hbm.at[c, sl])

  return body(edge_index, h, degp, zeros_nh)


def _tc_matmul_packed(x8, w1_big):
  def body(x_ref, w_ref, out_ref):
    h = jnp.dot(x_ref[...], w_ref[...], preferred_element_type=jnp.float32)
    out_ref[pl.ds(0, x_ref.shape[0]), :] = h

  return pl.pallas_call(
      body,
      out_shape=jax.ShapeDtypeStruct((NPAD // 8, 128), jnp.float32),
  )(x8, w1_big)


def _tc_final_packed(aggp2, b1t, wl_big, blt):
  half = NPAD * L // 128

  def body(a_ref, b1_ref, wl_ref, bl_ref, out_ref):
    v = a_ref[pl.ds(0, half), :] + a_ref[pl.ds(half, half), :] + b1_ref[...]
    act = jnp.maximum(v, 0.0)
    lg = jnp.dot(act, wl_ref[...], preferred_element_type=jnp.float32)
    lg = lg + bl_ref[...]
    out_ref[...] = 1.0 / (1.0 + jnp.exp(-lg))

  return pl.pallas_call(
      body,
      out_shape=jax.ShapeDtypeStruct((half, L), jnp.float32),
  )(aggp2, b1t, wl_big, blt)


def kernel(x_muons, edge_index_muons, W1, b1, Wl, bl, generate_jets=0):
  n, d = x_muons.shape
  h_dim = W1.shape[1]
  c_dim = Wl.shape[1]
  e = edge_index_muons.shape[1]
  e_per_tile = e // (NC * NS)

  eye8 = jnp.eye(8, dtype=jnp.float32)
  w1_big = jnp.einsum('jk,df->jdkf', eye8, W1).reshape(8 * d, 8 * h_dim)
  wl_big = jnp.einsum('jk,fc->jfkc', eye8, Wl).reshape(8 * h_dim, 8 * c_dim)
  b1t = jnp.tile(b1, 8).reshape(1, 8 * h_dim)
  blt = jnp.tile(bl, 8).reshape(1, 8 * c_dim)

  x8 = x_muons.reshape(n // 8, 8 * d)

  zeros_n = jnp.zeros((NPAD,), jnp.float32)
  ones_k = jnp.ones((KCH,), jnp.float32)
  zeros_nh = jnp.zeros((NPAD, h_dim), jnp.float32)

  h_pack = _tc_matmul_packed(x8, w1_big)
  degp = _sc_degree(edge_index_muons, zeros_n, ones_k, e_per_tile=e_per_tile)
  aggp, _ = _sc_aggregate(edge_index_muons, h_pack.reshape(NPAD, h_dim),
                          degp, zeros_nh, e_per_tile=e_per_tile)

  out_pack = _tc_final_packed(aggp.reshape(NC * NPAD * h_dim // 128, 128),
                              b1t, wl_big, blt)
  return out_pack.reshape(NPAD, c_dim)[:n]


# speedup vs baseline: 109.6990x; 1.2058x over previous
import functools

import jax
import jax.numpy as jnp
from jax import lax
from jax.experimental import pallas as pl
from jax.experimental.pallas import tpu as pltpu
from jax.experimental.pallas import tpu_sc as plsc

NC = 2
NS = 16
L = 16
NPAD = 10240
RPT = NPAD // NS
KCH = 2000


def _newton_rsqrt(d):
  i = lax.bitcast_convert_type(d, jnp.int32)
  i = 0x5F3759DF - lax.shift_right_arithmetic(i, 1)
  y = lax.bitcast_convert_type(i, jnp.float32)
  for _ in range(4):
    y = y * (1.5 - 0.5 * d * y * y)
  return y


_SPLAT_DNUMS = lax.GatherDimensionNumbers(
    offset_dims=(), collapsed_slice_dims=(0,), start_index_map=(0,))


def _splat(vec, jj):
  idx = jnp.full((L, 1), jj, jnp.int32)
  return lax.gather(vec, idx, _SPLAT_DNUMS, (1,),
                    mode=lax.GatherScatterMode.PROMISE_IN_BOUNDS)


@functools.partial(jax.jit, static_argnames=("e_per_tile",))
def _sc_degree(edge_index, zeros_n, ones_k, *, e_per_tile):
  mesh = plsc.VectorSubcoreMesh(core_axis_name="c", subcore_axis_name="s")
  nch = e_per_tile // KCH

  @functools.partial(
      pl.kernel,
      mesh=mesh,
      out_type=jax.ShapeDtypeStruct((NC, NPAD), jnp.float32),
      compiler_params=pltpu.CompilerParams(use_tc_tiling_on_sc=False),
      scratch_types=[
          pltpu.VMEM((nch, KCH), jnp.int32),
          pltpu.VMEM((KCH,), jnp.float32),
          pltpu.VMEM_SHARED((NPAD,), jnp.float32),
          pltpu.SemaphoreType.DMA,
          pltpu.SemaphoreType.DMA,
          pltpu.SemaphoreType.DMA,
      ],
  )
  def body(ei_hbm, zeros_hbm, ones_hbm, degp_hbm, idx_v, ones_v, acc_sh,
           isem, s0, s1):
    c = lax.axis_index("c")
    s = lax.axis_index("s")
    sl = pl.ds(s * RPT, RPT)
    base = (c * NS + s) * e_per_tile
    idma = [pltpu.async_copy(ei_hbm.at[1, pl.ds(base + i * KCH, KCH)],
                             idx_v.at[i], isem) for i in range(nch)]
    pltpu.sync_copy(zeros_hbm.at[sl], acc_sh.at[sl])
    pltpu.sync_copy(ones_hbm, ones_v)
    for d in idma:
      d.wait()
    plsc.subcore_barrier()
    ssems = [s0, s1]
    sdma = []
    for i in range(nch):
      if i >= 2:
        sdma[i - 2].wait()
      sdma.append(pltpu.async_copy(ones_v, acc_sh.at[idx_v.at[i]],
                                   ssems[i % 2], add=True))
    for d in sdma[-2:]:
      d.wait()
    plsc.subcore_barrier()
    pltpu.sync_copy(acc_sh.at[sl], degp_hbm.at[c, sl])

  return body(edge_index, zeros_n, ones_k)


@functools.partial(jax.jit, static_argnames=("e_per_tile",))
def _sc_aggregate(edge_index, h_pack, degp, zeros_nh, *, e_per_tile):
  mesh = plsc.VectorSubcoreMesh(core_axis_name="c", subcore_axis_name="s")
  nch = e_per_tile // KCH

  @functools.partial(
      pl.kernel,
      mesh=mesh,
      out_type=(jax.ShapeDtypeStruct((NC, NPAD, L), jnp.float32),
                jax.ShapeDtypeStruct((NPAD, L), jnp.float32)),
      compiler_params=pltpu.CompilerParams(use_tc_tiling_on_sc=False),
      scratch_types=[
          pltpu.VMEM((RPT,), jnp.float32),
          pltpu.VMEM((RPT,), jnp.float32),
          pltpu.VMEM((RPT // 8, 128), jnp.float32),
          pltpu.VMEM((RPT, L), jnp.float32),
          pltpu.VMEM((nch, KCH), jnp.int32),
          pltpu.VMEM((nch, KCH), jnp.int32),
          pltpu.VMEM((KCH, L), jnp.float32),
          pltpu.VMEM((KCH, L), jnp.float32),
          pltpu.VMEM_SHARED((NPAD, L), jnp.float32),
          pltpu.SemaphoreType.DMA,
          pltpu.SemaphoreType.DMA,
          pltpu.SemaphoreType.DMA,
          pltpu.SemaphoreType.DMA,
          pltpu.SemaphoreType.DMA,
      ],
  )
  def body(ei_hbm, h_hbm, degp_hbm, zeros_hbm, aggp_hbm, hs_hbm,
           d1_v, dinv_v, h_v, hs_v, sidx_v, didx_v, rows0_v, rows1_v,
           acc_sh, isem, g0, g1, sc0, sc1):
    c = lax.axis_index("c")
    s = lax.axis_index("s")
    sl = pl.ds(s * RPT, RPT)
    base = (c * NS + s) * e_per_tile

    idma = []
    for i in range(nch):
      ch = pl.ds(base + i * KCH, KCH)
      idma.append(pltpu.async_copy(ei_hbm.at[0, ch], sidx_v.at[i], isem))
      idma.append(pltpu.async_copy(ei_hbm.at[1, ch], didx_v.at[i], isem))

    pltpu.sync_copy(degp_hbm.at[0, sl], dinv_v)
    pltpu.sync_copy(degp_hbm.at[1, sl], d1_v)
    pltpu.sync_copy(h_hbm.at[pl.ds(s * (RPT // 8), RPT // 8)], h_v)

    def newton(g, carry):
      gsl = pl.ds(g * L, L)
      deg = dinv_v[gsl] + d1_v[gsl] + 1.0
      dinv_v[gsl] = _newton_rsqrt(deg)
      return carry

    lax.fori_loop(0, RPT // L, newton, 0)

    def scale_hs(g, carry):
      dchunk = dinv_v[pl.ds(g * L, L)]
      for jj in range(L):
        j = g * L + jj
        row = h_v[2 * g + jj // 8, pl.ds(16 * (jj % 8), L)]
        hs_v[j, :] = row * _splat(dchunk, jj)
      return carry

    lax.fori_loop(0, RPT // L, scale_hs, 0)
    pltpu.sync_copy(hs_v, hs_hbm.at[sl])

    @pl.when(c == 0)
    def _():
      pltpu.sync_copy(hs_v, acc_sh.at[sl])

    @pl.when(c != 0)
    def _():
      pltpu.sync_copy(zeros_hbm.at[sl], acc_sh.at[sl])

    for d in idma:
      d.wait()
    plsc.subcore_barrier()

    rows = [rows0_v, rows1_v]
    gsems = [g0, g1]
    ssems = [sc0, sc1]
    sdma = []
    for i in range(nch):
      b = i % 2
      if i >= 2:
        sdma[i - 2].wait()
      gd = pltpu.async_copy(hs_hbm.at[sidx_v.at[i]], rows[b], gsems[b])
      gd.wait()
      sdma.append(pltpu.async_copy(rows[b], acc_sh.at[didx_v.at[i]],
                                   ssems[b], add=True))
    for d in sdma[-2:]:
      d.wait()
    plsc.subcore_barrier()

    pltpu.sync_copy(acc_sh.at[sl], hs_v)

    def scale_out(g, carry):
      dchunk = dinv_v[pl.ds(g * L, L)]
      for jj in range(L):
        j = g * L + jj
        hs_v[j, :] = hs_v[j, :] * _splat(dchunk, jj)
      return carry

    lax.fori_loop(0, RPT // L, scale_out, 0)
    pltpu.sync_copy(hs_v, aggp_hbm.at[c, sl])

  return body(edge_index, h_pack, degp, zeros_nh)


def _tc_matmul_packed(x8, w1_big):
  def body(x_ref, w_ref, out_ref):
    h = jnp.dot(x_ref[...], w_ref[...], preferred_element_type=jnp.float32)
    out_ref[pl.ds(0, x_ref.shape[0]), :] = h

  return pl.pallas_call(
      body,
      out_shape=jax.ShapeDtypeStruct((NPAD // 8, 128), jnp.float32),
  )(x8, w1_big)


def _tc_final_packed(aggp2, b1t, wl_big, blt):
  half = NPAD * L // 128

  def body(a_ref, b1_ref, wl_ref, bl_ref, out_ref):
    v = a_ref[pl.ds(0, half), :] + a_ref[pl.ds(half, half), :] + b1_ref[...]
    act = jnp.maximum(v, 0.0)
    lg = jnp.dot(act, wl_ref[...], preferred_element_type=jnp.float32)
    out_ref[...] = 1.0 / (1.0 + jnp.exp(-(lg + bl_ref[...])))

  return pl.pallas_call(
      body,
      out_shape=jax.ShapeDtypeStruct((half, L), jnp.float32),
  )(aggp2, b1t, wl_big, blt)


def kernel(x_muons, edge_index_muons, W1, b1, Wl, bl, generate_jets=0):
  n, d = x_muons.shape
  h_dim = W1.shape[1]
  c_dim = Wl.shape[1]
  e = edge_index_muons.shape[1]
  e_per_tile = e // (NC * NS)

  eye8 = jnp.eye(8, dtype=jnp.float32)
  w1_big = jnp.einsum('jk,df->jdkf', eye8, W1).reshape(8 * d, 8 * h_dim)
  wl_big = jnp.einsum('jk,fc->jfkc', eye8, Wl).reshape(8 * h_dim, 8 * c_dim)
  b1t = jnp.tile(b1, 8).reshape(1, 8 * h_dim)
  blt = jnp.tile(bl, 8).reshape(1, 8 * c_dim)

  zeros_n = jnp.zeros((NPAD,), jnp.float32)
  ones_k = jnp.ones((KCH,), jnp.float32)
  zeros_nh = jnp.zeros((NPAD, h_dim), jnp.float32)

  x8 = x_muons.reshape(n // 8, 8 * d)
  h_pack = _tc_matmul_packed(x8, w1_big)
  degp = _sc_degree(edge_index_muons, zeros_n, ones_k, e_per_tile=e_per_tile)
  aggp, _ = _sc_aggregate(edge_index_muons, h_pack, degp, zeros_nh,
                          e_per_tile=e_per_tile)

  out_pack = _tc_final_packed(aggp.reshape(NC * NPAD * h_dim // 128, 128),
                              b1t, wl_big, blt)
  return out_pack.reshape(NPAD, c_dim)[:n]


# speedup vs baseline: 117.4120x; 1.0703x over previous
import functools

import jax
import jax.numpy as jnp
from jax import lax
from jax.experimental import pallas as pl
from jax.experimental.pallas import tpu as pltpu
from jax.experimental.pallas import tpu_sc as plsc

NC = 2
NS = 16
L = 16
NPAD = 10240
RPT = NPAD // NS
KDEG = 2000
KAGG = 1000
NBUF = 4


def _newton_rsqrt(d):
  i = lax.bitcast_convert_type(d, jnp.int32)
  i = 0x5F3759DF - lax.shift_right_arithmetic(i, 1)
  y = lax.bitcast_convert_type(i, jnp.float32)
  for _ in range(4):
    y = y * (1.5 - 0.5 * d * y * y)
  return y


_SPLAT_DNUMS = lax.GatherDimensionNumbers(
    offset_dims=(), collapsed_slice_dims=(0,), start_index_map=(0,))


def _splat(vec, jj):
  idx = jnp.full((L, 1), jj, jnp.int32)
  return lax.gather(vec, idx, _SPLAT_DNUMS, (1,),
                    mode=lax.GatherScatterMode.PROMISE_IN_BOUNDS)


@functools.partial(jax.jit, static_argnames=("e_per_tile",))
def _sc_degree(edge_index, *, e_per_tile):
  mesh = plsc.VectorSubcoreMesh(core_axis_name="c", subcore_axis_name="s")
  nch = e_per_tile // KDEG

  @functools.partial(
      pl.kernel,
      mesh=mesh,
      out_type=jax.ShapeDtypeStruct((NC, NPAD), jnp.float32),
      compiler_params=pltpu.CompilerParams(use_tc_tiling_on_sc=False),
      scratch_types=[
          pltpu.VMEM((nch, KDEG), jnp.int32),
          pltpu.VMEM((KDEG,), jnp.float32),
          pltpu.VMEM_SHARED((NPAD,), jnp.float32),
          pltpu.SemaphoreType.DMA,
          pltpu.SemaphoreType.DMA,
          pltpu.SemaphoreType.DMA,
      ],
  )
  def body(ei_hbm, degp_hbm, idx_v, ones_v, acc_sh, isem, s0, s1):
    c = lax.axis_index("c")
    s = lax.axis_index("s")
    sl = pl.ds(s * RPT, RPT)
    base = (c * NS + s) * e_per_tile
    idma = [pltpu.async_copy(ei_hbm.at[1, pl.ds(base + i * KDEG, KDEG)],
                             idx_v.at[i], isem) for i in range(nch)]

    def fill_zero(i, carry):
      ones_v[pl.ds(i * L, L)] = jnp.zeros((L,), jnp.float32)
      return carry

    lax.fori_loop(0, RPT // L, fill_zero, 0)
    pltpu.sync_copy(ones_v.at[pl.ds(0, RPT)], acc_sh.at[sl])

    def fill_one(i, carry):
      ones_v[pl.ds(i * L, L)] = jnp.ones((L,), jnp.float32)
      return carry

    lax.fori_loop(0, KDEG // L, fill_one, 0)
    for d in idma:
      d.wait()
    plsc.subcore_barrier()
    ssems = [s0, s1]
    sdma = []
    for i in range(nch):
      if i >= 2:
        sdma[i - 2].wait()
      sdma.append(pltpu.async_copy(ones_v, acc_sh.at[idx_v.at[i]],
                                   ssems[i % 2], add=True))
    for d in sdma[-2:]:
      d.wait()
    plsc.subcore_barrier()
    pltpu.sync_copy(acc_sh.at[sl], degp_hbm.at[c, sl])

  return body(edge_index)


@functools.partial(jax.jit, static_argnames=("e_per_tile",))
def _sc_aggregate(edge_index, h_pack, degp, *, e_per_tile):
  mesh = plsc.VectorSubcoreMesh(core_axis_name="c", subcore_axis_name="s")
  nch = e_per_tile // KAGG

  @functools.partial(
      pl.kernel,
      mesh=mesh,
      out_type=(jax.ShapeDtypeStruct((NC, NPAD, L), jnp.float32),
                jax.ShapeDtypeStruct((NPAD, L), jnp.float32)),
      compiler_params=pltpu.CompilerParams(use_tc_tiling_on_sc=False),
      scratch_types=[
          pltpu.VMEM((RPT,), jnp.float32),
          pltpu.VMEM((RPT,), jnp.float32),
          pltpu.VMEM((RPT // 8, 128), jnp.float32),
          pltpu.VMEM((RPT, L), jnp.float32),
          pltpu.VMEM((nch, KAGG), jnp.int32),
          pltpu.VMEM((nch, KAGG), jnp.int32),
          [pltpu.VMEM((KAGG, L), jnp.float32) for _ in range(NBUF)],
          pltpu.VMEM_SHARED((NPAD, L), jnp.float32),
          pltpu.SemaphoreType.DMA,
          [pltpu.SemaphoreType.DMA for _ in range(NBUF)],
          [pltpu.SemaphoreType.DMA for _ in range(NBUF)],
      ],
  )
  def body(ei_hbm, h_hbm, degp_hbm, aggp_hbm, hs_hbm,
           d1_v, dinv_v, h_v, hs_v, sidx_v, didx_v, rows, acc_sh,
           isem, gsems, ssems):
    c = lax.axis_index("c")
    s = lax.axis_index("s")
    sl = pl.ds(s * RPT, RPT)
    base = (c * NS + s) * e_per_tile

    idma = []
    for i in range(nch):
      ch = pl.ds(base + i * KAGG, KAGG)
      idma.append(pltpu.async_copy(ei_hbm.at[0, ch], sidx_v.at[i], isem))
      idma.append(pltpu.async_copy(ei_hbm.at[1, ch], didx_v.at[i], isem))

    pltpu.sync_copy(degp_hbm.at[0, sl], dinv_v)
    pltpu.sync_copy(degp_hbm.at[1, sl], d1_v)
    pltpu.sync_copy(h_hbm.at[pl.ds(s * (RPT // 8), RPT // 8)], h_v)

    def newton(g, carry):
      gsl = pl.ds(g * L, L)
      deg = dinv_v[gsl] + d1_v[gsl] + 1.0
      dinv_v[gsl] = _newton_rsqrt(deg)
      return carry

    lax.fori_loop(0, RPT // L, newton, 0)

    def scale_hs(g, carry):
      dchunk = dinv_v[pl.ds(g * L, L)]
      for jj in range(L):
        j = g * L + jj
        row = h_v[2 * g + jj // 8, pl.ds(16 * (jj % 8), L)]
        hs_v[j, :] = row * _splat(dchunk, jj)
      return carry

    lax.fori_loop(0, RPT // L, scale_hs, 0)
    pltpu.sync_copy(hs_v, hs_hbm.at[sl])

    @pl.when(c == 0)
    def _():
      pltpu.sync_copy(hs_v, acc_sh.at[sl])

    @pl.when(c != 0)
    def _():
      def fill_zero(i, carry):
        rows[0][i, :] = jnp.zeros((L,), jnp.float32)
        return carry

      lax.fori_loop(0, RPT, fill_zero, 0)
      pltpu.sync_copy(rows[0].at[pl.ds(0, RPT)], acc_sh.at[sl])

    for d in idma:
      d.wait()
    plsc.subcore_barrier()

    sdma = {}
    gdma = {}

    def issue_gather(i):
      b = i % NBUF
      if i - NBUF >= 0:
        sdma[i - NBUF].wait()
      gdma[i] = pltpu.async_copy(hs_hbm.at[sidx_v.at[i]], rows[b], gsems[b])

    issue_gather(0)
    if nch > 1:
      issue_gather(1)
    for i in range(nch):
      b = i % NBUF
      gdma[i].wait()
      sdma[i] = pltpu.async_copy(rows[b], acc_sh.at[didx_v.at[i]],
                                 ssems[b], add=True)
      if i + 2 < nch:
        issue_gather(i + 2)
    for i in range(max(0, nch - NBUF), nch):
      sdma[i].wait()
    plsc.subcore_barrier()

    pltpu.sync_copy(acc_sh.at[sl], hs_v)

    def scale_out(g, carry):
      dchunk = dinv_v[pl.ds(g * L, L)]
      for jj in range(L):
        j = g * L + jj
        hs_v[j, :] = hs_v[j, :] * _splat(dchunk, jj)
      return carry

    lax.fori_loop(0, RPT // L, scale_out, 0)
    pltpu.sync_copy(hs_v, aggp_hbm.at[c, sl])

  return body(edge_index, h_pack, degp)


def _tc_matmul_packed(x3, w1):
  n8 = x3.shape[0]

  def body(x_ref, w_ref, out_ref):
    xa = x_ref[...]
    w = w_ref[...]
    for j in range(8):
      xj = xa[:, j, :]
      hj = jnp.dot(xj, w, preferred_element_type=jnp.float32)
      out_ref[pl.ds(0, n8), pl.ds(L * j, L)] = hj

  return pl.pallas_call(
      body,
      out_shape=jax.ShapeDtypeStruct((NPAD // 8, 128), jnp.float32),
  )(x3, w1)


def _tc_final_packed(aggp2, b1, wl, bl):
  half = NPAD * L // 128
  h_dim = wl.shape[0]
  c_dim = wl.shape[1]

  def body(a_ref, b1_ref, wl_ref, bl_ref, out_ref):
    b1t = jnp.concatenate([b1_ref[...]] * 8, axis=1)
    blt = jnp.concatenate([bl_ref[...]] * 8, axis=1)
    w = wl_ref[...]
    z = jnp.zeros((h_dim, c_dim), jnp.float32)
    wl_big = jnp.concatenate(
        [jnp.concatenate([w if k == j else z for k in range(8)], axis=0)
         for j in range(8)], axis=1)
    v = a_ref[pl.ds(0, half), :] + a_ref[pl.ds(half, half), :] + b1t
    act = jnp.maximum(v, 0.0)
    lg = jnp.dot(act, wl_big, preferred_element_type=jnp.float32)
    out_ref[...] = 1.0 / (1.0 + jnp.exp(-(lg + blt)))

  return pl.pallas_call(
      body,
      out_shape=jax.ShapeDtypeStruct((half, L), jnp.float32),
  )(aggp2, b1, wl, bl)


def kernel(x_muons, edge_index_muons, W1, b1, Wl, bl, generate_jets=0):
  n, d = x_muons.shape
  h_dim = W1.shape[1]
  c_dim = Wl.shape[1]
  e = edge_index_muons.shape[1]
  e_per_tile = e // (NC * NS)

  x3 = x_muons.reshape(n // 8, 8, d)

  h_pack = _tc_matmul_packed(x3, W1)
  degp = _sc_degree(edge_index_muons, e_per_tile=e_per_tile)
  aggp, _ = _sc_aggregate(edge_index_muons, h_pack, degp,
                          e_per_tile=e_per_tile)

  out_pack = _tc_final_packed(aggp.reshape(NC * NPAD * h_dim // 128, 128),
                              b1.reshape(1, h_dim), Wl,
                              bl.reshape(1, c_dim))
  return out_pack[:n // 8].reshape(n, c_dim)


# speedup vs baseline: 124.4448x; 1.0599x over previous
import functools

import jax
import jax.numpy as jnp
from jax import lax
from jax.experimental import pallas as pl
from jax.experimental.pallas import tpu as pltpu
from jax.experimental.pallas import tpu_sc as plsc

NC = 2
NS = 16
L = 16
NPAD = 10240
RPT = NPAD // NS
KDEG = 2000
KAGG = 1000
NBUF = 4


def _newton_rsqrt(d):
  i = lax.bitcast_convert_type(d, jnp.int32)
  i = 0x5F3759DF - lax.shift_right_arithmetic(i, 1)
  y = lax.bitcast_convert_type(i, jnp.float32)
  for _ in range(4):
    y = y * (1.5 - 0.5 * d * y * y)
  return y


_SPLAT_DNUMS = lax.GatherDimensionNumbers(
    offset_dims=(), collapsed_slice_dims=(0,), start_index_map=(0,))


def _splat(vec, jj):
  idx = jnp.full((L, 1), jj, jnp.int32)
  return lax.gather(vec, idx, _SPLAT_DNUMS, (1,),
                    mode=lax.GatherScatterMode.PROMISE_IN_BOUNDS)


@functools.partial(jax.jit, static_argnames=("e_per_tile",))
def _sc_degree(edge_index, *, e_per_tile):
  mesh = plsc.VectorSubcoreMesh(core_axis_name="c", subcore_axis_name="s")
  nch = e_per_tile // KDEG

  @functools.partial(
      pl.kernel,
      mesh=mesh,
      out_type=jax.ShapeDtypeStruct((NC, NPAD), jnp.float32),
      compiler_params=pltpu.CompilerParams(use_tc_tiling_on_sc=False),
      scratch_types=[
          pltpu.VMEM((nch, KDEG), jnp.int32),
          pltpu.VMEM((KDEG,), jnp.float32),
          pltpu.VMEM_SHARED((NPAD,), jnp.float32),
          pltpu.SemaphoreType.DMA,
          pltpu.SemaphoreType.DMA,
          pltpu.SemaphoreType.DMA,
      ],
  )
  def body(ei_hbm, degp_hbm, idx_v, ones_v, acc_sh, isem, s0, s1):
    c = lax.axis_index("c")
    s = lax.axis_index("s")
    sl = pl.ds(s * RPT, RPT)
    base = (c * NS + s) * e_per_tile
    idma = [pltpu.async_copy(ei_hbm.at[1, pl.ds(base + i * KDEG, KDEG)],
                             idx_v.at[i], isem) for i in range(nch)]

    def fill_zero(i, carry):
      ones_v[pl.ds(i * L, L)] = jnp.zeros((L,), jnp.float32)
      return carry

    lax.fori_loop(0, RPT // L, fill_zero, 0)
    pltpu.sync_copy(ones_v.at[pl.ds(0, RPT)], acc_sh.at[sl])

    def fill_one(i, carry):
      ones_v[pl.ds(i * L, L)] = jnp.ones((L,), jnp.float32)
      return carry

    lax.fori_loop(0, KDEG // L, fill_one, 0)
    for d in idma:
      d.wait()
    plsc.subcore_barrier()
    ssems = [s0, s1]
    sdma = []
    for i in range(nch):
      if i >= 2:
        sdma[i - 2].wait()
      sdma.append(pltpu.async_copy(ones_v, acc_sh.at[idx_v.at[i]],
                                   ssems[i % 2], add=True))
    for d in sdma[-2:]:
      d.wait()
    plsc.subcore_barrier()
    pltpu.sync_copy(acc_sh.at[sl], degp_hbm.at[c, sl])

  return body(edge_index)


@functools.partial(jax.jit, static_argnames=("e_per_tile",))
def _sc_aggregate(edge_index, h_pack, degp, *, e_per_tile):
  mesh = plsc.VectorSubcoreMesh(core_axis_name="c", subcore_axis_name="s")
  nch = e_per_tile // KAGG

  @functools.partial(
      pl.kernel,
      mesh=mesh,
      out_type=jax.ShapeDtypeStruct((NC, NPAD, L), jnp.float32),
      compiler_params=pltpu.CompilerParams(use_tc_tiling_on_sc=False),
      scratch_types=[
          pltpu.VMEM((RPT,), jnp.float32),
          pltpu.VMEM((RPT,), jnp.float32),
          pltpu.VMEM((RPT // 8, 128), jnp.float32),
          pltpu.VMEM((RPT, L), jnp.float32),
          pltpu.VMEM((nch, KAGG), jnp.int32),
          pltpu.VMEM((nch, KAGG), jnp.int32),
          [pltpu.VMEM((KAGG, L), jnp.float32) for _ in range(NBUF)],
          pltpu.VMEM_SHARED((NPAD, L), jnp.float32),
          pltpu.VMEM_SHARED((NPAD, L), jnp.float32),
          pltpu.SemaphoreType.DMA,
          [pltpu.SemaphoreType.DMA for _ in range(NBUF)],
          [pltpu.SemaphoreType.DMA for _ in range(NBUF)],
      ],
  )
  def body(ei_hbm, h_hbm, degp_hbm, aggp_hbm,
           d1_v, dinv_v, h_v, hs_v, sidx_v, didx_v, rows, acc_sh, hs_sh,
           isem, gsems, ssems):
    c = lax.axis_index("c")
    s = lax.axis_index("s")
    sl = pl.ds(s * RPT, RPT)
    base = (c * NS + s) * e_per_tile

    idma = []
    for i in range(nch):
      ch = pl.ds(base + i * KAGG, KAGG)
      idma.append(pltpu.async_copy(ei_hbm.at[0, ch], sidx_v.at[i], isem))
      idma.append(pltpu.async_copy(ei_hbm.at[1, ch], didx_v.at[i], isem))

    pltpu.sync_copy(degp_hbm.at[0, sl], dinv_v)
    pltpu.sync_copy(degp_hbm.at[1, sl], d1_v)
    pltpu.sync_copy(h_hbm.at[pl.ds(s * (RPT // 8), RPT // 8)], h_v)

    def newton(g, carry):
      gsl = pl.ds(g * L, L)
      deg = dinv_v[gsl] + d1_v[gsl] + 1.0
      dinv_v[gsl] = _newton_rsqrt(deg)
      return carry

    lax.fori_loop(0, RPT // L, newton, 0)

    def scale_hs(g, carry):
      dchunk = dinv_v[pl.ds(g * L, L)]
      for jj in range(L):
        j = g * L + jj
        row = h_v[2 * g + jj // 8, pl.ds(16 * (jj % 8), L)]
        hs_v[j, :] = row * _splat(dchunk, jj)
      return carry

    lax.fori_loop(0, RPT // L, scale_hs, 0)
    pltpu.sync_copy(hs_v, hs_sh.at[sl])

    @pl.when(c == 0)
    def _():
      pltpu.sync_copy(hs_v, acc_sh.at[sl])

    @pl.when(c != 0)
    def _():
      def fill_zero(i, carry):
        rows[0][i, :] = jnp.zeros((L,), jnp.float32)
        return carry

      lax.fori_loop(0, RPT, fill_zero, 0)
      pltpu.sync_copy(rows[0].at[pl.ds(0, RPT)], acc_sh.at[sl])

    for d in idma:
      d.wait()
    plsc.subcore_barrier()

    sdma = {}
    gdma = {}

    def issue_gather(i):
      b = i % NBUF
      if i - NBUF >= 0:
        sdma[i - NBUF].wait()
      gdma[i] = pltpu.async_copy(hs_sh.at[sidx_v.at[i]], rows[b], gsems[b])

    issue_gather(0)
    if nch > 1:
      issue_gather(1)
    for i in range(nch):
      b = i % NBUF
      gdma[i].wait()
      sdma[i] = pltpu.async_copy(rows[b], acc_sh.at[didx_v.at[i]],
                                 ssems[b], add=True)
      if i + 2 < nch:
        issue_gather(i + 2)
    for i in range(max(0, nch - NBUF), nch):
      sdma[i].wait()
    plsc.subcore_barrier()

    pltpu.sync_copy(acc_sh.at[sl], hs_v)

    def scale_out(g, carry):
      dchunk = dinv_v[pl.ds(g * L, L)]
      for jj in range(L):
        j = g * L + jj
        hs_v[j, :] = hs_v[j, :] * _splat(dchunk, jj)
      return carry

    lax.fori_loop(0, RPT // L, scale_out, 0)
    pltpu.sync_copy(hs_v, aggp_hbm.at[c, sl])

  return body(edge_index, h_pack, degp)


def _tc_matmul_packed(x3, w1):
  n8 = x3.shape[0]

  def body(x_ref, w_ref, out_ref):
    xa = x_ref[...]
    w = w_ref[...]
    for j in range(8):
      xj = xa[:, j, :]
      hj = jnp.dot(xj, w, preferred_element_type=jnp.float32)
      out_ref[pl.ds(0, n8), pl.ds(L * j, L)] = hj

  return pl.pallas_call(
      body,
      out_shape=jax.ShapeDtypeStruct((NPAD // 8, 128), jnp.float32),
  )(x3, w1)


def _tc_final_packed(aggp2, b1, wl, bl, n8):
  half = NPAD * L // 128
  h_dim = wl.shape[0]
  c_dim = wl.shape[1]

  def body(a_ref, b1_ref, wl_ref, bl_ref, out_ref):
    b1t = jnp.concatenate([b1_ref[...]] * 8, axis=1)
    blt = jnp.concatenate([bl_ref[...]] * 8, axis=1)
    w = wl_ref[...]
    z = jnp.zeros((h_dim, c_dim), jnp.float32)
    wl_big = jnp.concatenate(
        [jnp.concatenate([w if k == j else z for k in range(8)], axis=0)
         for j in range(8)], axis=1)
    v = a_ref[pl.ds(0, half), :] + a_ref[pl.ds(half, half), :] + b1t
    act = jnp.maximum(v, 0.0)
    lg = jnp.dot(act, wl_big, preferred_element_type=jnp.float32)
    sg = 1.0 / (1.0 + jnp.exp(-(lg + blt)))
    out_ref[...] = sg[:out_ref.shape[0]]

  return pl.pallas_call(
      body,
      out_shape=jax.ShapeDtypeStruct((n8, L), jnp.float32),
  )(aggp2, b1, wl, bl)


def kernel(x_muons, edge_index_muons, W1, b1, Wl, bl, generate_jets=0):
  n, d = x_muons.shape
  h_dim = W1.shape[1]
  c_dim = Wl.shape[1]
  e = edge_index_muons.shape[1]
  e_per_tile = e // (NC * NS)

  x3 = x_muons.reshape(n // 8, 8, d)

  h_pack = _tc_matmul_packed(x3, W1)
  degp = _sc_degree(edge_index_muons, e_per_tile=e_per_tile)
  aggp = _sc_aggregate(edge_index_muons, h_pack, degp,
                       e_per_tile=e_per_tile)

  out_pack = _tc_final_packed(aggp.reshape(NC * NPAD * h_dim // 128, 128),
                              b1.reshape(1, h_dim), Wl,
                              bl.reshape(1, c_dim), n // 8)
  return out_pack.reshape(n, c_dim)
